# Initial kernel scaffold; baseline (speedup 1.0000x reference)
import jax
import jax.numpy as jnp
from jax.experimental import pallas as pl


def kernel(x, edge_index, batch, W1, b1, W2, b2, g1, be1, W3, b3, W4, b4, g2, be2, Wf, bf):
    raise NotImplementedError("write your pallas kernel here")


# speedup vs baseline: 4.8567x; 4.8567x over previous
import functools

import jax
import jax.numpy as jnp
from jax import lax
from jax.experimental import pallas as pl
from jax.experimental.pallas import tpu as pltpu
from jax.experimental.pallas import tpu_sc as plsc

_N = 10000
_E = 320000
_D = 128
_G = 64
_EPS = 1e-5

_NC = 2
_NS = 16
_NW = _NC * _NS
_EPW = _E // _NW
_K = 80
_NCHUNK = _EPW // _K
_RPT = 624
_ZR = 208


def _segsum_body(x_hbm, src_hbm, dst_hbm, out_hbm, acc, zbuf, src_v, dst_v,
                 rows_v, sem):
  cid = lax.axis_index("c")
  sid = lax.axis_index("s")
  wid = sid * _NC + cid

  @pl.loop(0, _ZR)
  def _(i):
    @pl.loop(0, _D // 16)
    def _(j):
      zbuf[i, pl.ds(j * 16, 16)] = jnp.zeros((16,), jnp.float32)

  @pl.loop(0, _RPT // _ZR)
  def _(i):
    off = pl.multiple_of(sid * _RPT + i * _ZR, 8)
    pltpu.sync_copy(zbuf, acc.at[pl.ds(off, _ZR)])

  @pl.when(sid == _NS - 1)
  def _():
    pltpu.sync_copy(zbuf.at[pl.ds(0, _N - _NS * _RPT)],
                    acc.at[pl.ds(_NS * _RPT, _N - _NS * _RPT)])

  plsc.subcore_barrier()

  base = wid * _EPW

  @pl.loop(0, _NCHUNK)
  def _(i):
    off = base + i * _K
    pltpu.sync_copy(src_hbm.at[pl.ds(off, _K)], src_v)
    pltpu.sync_copy(dst_hbm.at[pl.ds(off, _K)], dst_v)
    pltpu.async_copy(x_hbm.at[src_v], rows_v, sem).wait()
    pltpu.sync_copy(rows_v, acc.at[dst_v], add=True)

  plsc.subcore_barrier()

  @pl.loop(0, _RPT // _ZR)
  def _(i):
    off = pl.multiple_of(sid * _RPT + i * _ZR, 8)
    pltpu.sync_copy(acc.at[pl.ds(off, _ZR)],
                    out_hbm.at[pl.ds(cid * _N + off, _ZR)])

  @pl.when(sid == _NS - 1)
  def _():
    pltpu.sync_copy(acc.at[pl.ds(_NS * _RPT, _N - _NS * _RPT)],
                    out_hbm.at[pl.ds(cid * _N + _NS * _RPT, _N - _NS * _RPT)])


@functools.cache
def _get_segsum():
  return pl.kernel(
      _segsum_body,
      out_type=jax.ShapeDtypeStruct((_NC * _N, _D), jnp.float32),
      mesh=plsc.VectorSubcoreMesh(core_axis_name="c", subcore_axis_name="s",
                                  num_cores=_NC, num_subcores=_NS),
      scratch_types=[
          pltpu.VMEM_SHARED((_N, _D), jnp.float32),
          pltpu.VMEM((_ZR, _D), jnp.float32),
          pltpu.VMEM((_K,), jnp.int32),
          pltpu.VMEM((_K,), jnp.int32),
          pltpu.VMEM((_K, _D), jnp.float32),
          pltpu.SemaphoreType.DMA,
      ],
  )


def _dense_body(x_ref, p_ref, Wa_ref, ba_ref, Wb_ref, bb_ref, g_ref, be_ref,
                o_ref):
  h0 = x_ref[...] + p_ref[0:_N, :] + p_ref[_N:2 * _N, :]
  t = jnp.dot(h0, Wa_ref[...], preferred_element_type=jnp.float32)
  t = jnp.maximum(t + ba_ref[...], 0.0)
  h = jnp.dot(t, Wb_ref[...], preferred_element_type=jnp.float32) + bb_ref[...]
  m = jnp.mean(h, axis=0, keepdims=True)
  c = h - m
  v = jnp.mean(c * c, axis=0, keepdims=True)
  hn = c * lax.rsqrt(v + _EPS) * g_ref[...] + be_ref[...]
  o_ref[...] = jnp.maximum(hn, 0.0)


_dense = pl.pallas_call(
    _dense_body,
    out_shape=jax.ShapeDtypeStruct((_N, _D), jnp.float32),
)


def _pool_body(h_ref, batch_ref, Wf_ref, bf_ref, o_ref):
  gids = lax.broadcasted_iota(jnp.int32, (_G, _N), 0)
  mask = (gids == batch_ref[...]).astype(jnp.float32)
  sums = jnp.dot(mask, h_ref[...], preferred_element_type=jnp.float32)
  counts = jnp.sum(mask, axis=1, keepdims=True)
  pooled = sums / jnp.maximum(counts, 1.0)
  o_ref[...] = (
      jnp.dot(pooled, Wf_ref[...], preferred_element_type=jnp.float32)
      + bf_ref[...])


_pool = pl.pallas_call(
    _pool_body,
    out_shape=jax.ShapeDtypeStruct((_G, _D), jnp.float32),
)


@jax.jit
def kernel(x, edge_index, batch, W1, b1, W2, b2, g1, be1, W3, b3, W4, b4, g2,
           be2, Wf, bf):
  src = edge_index[0]
  dst = edge_index[1]
  _segsum = _get_segsum()
  p1 = _segsum(x, src, dst)
  h1 = _dense(x, p1, W1, b1.reshape(1, _D), W2, b2.reshape(1, _D),
              g1.reshape(1, _D), be1.reshape(1, _D))
  p2 = _segsum(h1, src, dst)
  h2 = _dense(h1, p2, W3, b3.reshape(1, _D), W4, b4.reshape(1, _D),
              g2.reshape(1, _D), be2.reshape(1, _D))
  return _pool(h2, batch.reshape(1, _N), Wf, bf.reshape(1, _D))
